# baseline (speedup 1.0000x reference)
import functools

import jax
import jax.numpy as jnp
from jax import lax
from jax.experimental import pallas as pl
from jax.experimental.pallas import tpu as pltpu
from jax.experimental.pallas import tpu_sc as plsc

N_NODES = 10000
N_EDGES = 320000
D = 128

NUM_CORES = 2
NUM_SUBCORES = 16
NUM_WORKERS = NUM_CORES * NUM_SUBCORES
BLK = 40
NBLK = 252
EDGES_PAD = NUM_WORKERS * NBLK * BLK
ACC_ROWS = 10112
ROWS_PER_TILE = ACC_ROWS // NUM_SUBCORES


def _sc_agg_body(h_hbm, src_hbm, dst_hbm, zeros_hbm, out_hbm,
                 sidx0, didx0, sidx1, didx1, rows0, rows1,
                 acc, g0, g1, si0, si1):
    cid = lax.axis_index("c")
    sid = lax.axis_index("s")
    wid = cid * NUM_SUBCORES + sid
    base0 = wid * (NBLK * BLK)

    sidx = (sidx0, sidx1)
    didx = (didx0, didx1)
    rows = (rows0, rows1)
    g = (g0, g1)
    si = (si0, si1)

    def idx_load(j, p):
        pltpu.async_copy(src_hbm.at[pl.ds(base0 + j * BLK, BLK)], sidx[p], si[p])
        pltpu.async_copy(dst_hbm.at[pl.ds(base0 + j * BLK, BLK)], didx[p], si[p])

    def idx_wait(j, p):
        pltpu.make_async_copy(
            src_hbm.at[pl.ds(base0 + j * BLK, BLK)], sidx[p], si[p]).wait()
        pltpu.make_async_copy(
            dst_hbm.at[pl.ds(base0 + j * BLK, BLK)], didx[p], si[p]).wait()

    def gather(p):
        pltpu.async_copy(h_hbm.at[sidx[p]], rows[p], g[p])

    def gather_wait(p):
        pltpu.make_async_copy(h_hbm.at[sidx[p]], rows[p], g[p]).wait()

    def scatter(p):
        pltpu.sync_copy(rows[p], acc.at[didx[p]], add=True)

    idx_load(0, 0)
    idx_load(1, 1)
    pltpu.sync_copy(zeros_hbm, acc.at[pl.ds(sid * ROWS_PER_TILE, ROWS_PER_TILE)])
    idx_wait(0, 0)
    plsc.subcore_barrier()
    gather(0)

    @pl.loop(0, NBLK - 2, step=2)
    def _(j):
        gather_wait(0)
        scatter(0)
        idx_load(j + 2, 0)
        idx_wait(j + 1, 1)
        gather(1)

        gather_wait(1)
        scatter(1)
        idx_load(j + 3, 1)
        idx_wait(j + 2, 0)
        gather(0)

    gather_wait(0)
    scatter(0)
    idx_wait(NBLK - 1, 1)
    gather(1)
    gather_wait(1)
    scatter(1)

    plsc.subcore_barrier()
    row0 = sid * ROWS_PER_TILE
    pltpu.sync_copy(acc.at[pl.ds(row0, ROWS_PER_TILE)],
                    out_hbm.at[cid, pl.ds(row0, ROWS_PER_TILE)])


@jax.jit
def _sc_agg(h, src, dst, zeros):
    mesh = plsc.VectorSubcoreMesh(core_axis_name="c", subcore_axis_name="s")
    k = pl.kernel(
        _sc_agg_body,
        out_type=jax.ShapeDtypeStruct((NUM_CORES, ACC_ROWS, D), jnp.float32),
        mesh=mesh,
        scratch_types=[
            pltpu.VMEM((BLK,), jnp.int32),
            pltpu.VMEM((BLK,), jnp.int32),
            pltpu.VMEM((BLK,), jnp.int32),
            pltpu.VMEM((BLK,), jnp.int32),
            pltpu.VMEM((BLK, D), jnp.float32),
            pltpu.VMEM((BLK, D), jnp.float32),
            pltpu.VMEM_SHARED((ACC_ROWS, D), jnp.float32),
            pltpu.SemaphoreType.DMA,
            pltpu.SemaphoreType.DMA,
            pltpu.SemaphoreType.DMA,
            pltpu.SemaphoreType.DMA,
        ],
    )
    return k(h, src, dst, zeros)


ROW_BLK = 1000


def _tc_mlp_body(x_ref, p_ref, wt_ref, b_ref, o_ref, *, relu):
    s = x_ref[...] + p_ref[0] + p_ref[1]
    y = jnp.dot(s, wt_ref[...], preferred_element_type=jnp.float32) + b_ref[...]
    if relu:
        y = jnp.maximum(y, 0.0)
    o_ref[...] = y


def _tc_mlp(x, parts, wt, b, relu):
    grid = (N_NODES // ROW_BLK,)
    return pl.pallas_call(
        functools.partial(_tc_mlp_body, relu=relu),
        grid=grid,
        in_specs=[
            pl.BlockSpec((ROW_BLK, D), lambda i: (i, 0)),
            pl.BlockSpec((NUM_CORES, ROW_BLK, D), lambda i: (0, i, 0)),
            pl.BlockSpec((D, D), lambda i: (0, 0)),
            pl.BlockSpec((1, D), lambda i: (0, 0)),
        ],
        out_specs=pl.BlockSpec((ROW_BLK, D), lambda i: (i, 0)),
        out_shape=jax.ShapeDtypeStruct((N_NODES, D), jnp.float32),
    )(x, parts, wt, b)


def kernel(x, edge_index, W1, b1, W2, b2):
    npad = EDGES_PAD - N_EDGES
    src = jnp.concatenate(
        [edge_index[0].astype(jnp.int32), jnp.zeros((npad,), jnp.int32)])
    dst = jnp.concatenate(
        [edge_index[1].astype(jnp.int32),
         jnp.full((npad,), ACC_ROWS - 1, jnp.int32)])
    zeros = jnp.zeros((ROWS_PER_TILE, D), jnp.float32)

    agg1 = _sc_agg(x, src, dst, zeros)
    h = _tc_mlp(x, agg1, W1.T, b1.reshape(1, D), relu=True)
    agg2 = _sc_agg(h, src, dst, zeros)
    out = _tc_mlp(h, agg2, W2.T, b2.reshape(1, D), relu=False)
    return out


# speedup vs baseline: 1.4362x; 1.4362x over previous
import functools

import jax
import jax.numpy as jnp
from jax import lax
from jax.experimental import pallas as pl
from jax.experimental.pallas import tpu as pltpu
from jax.experimental.pallas import tpu_sc as plsc

N_NODES = 10000
N_EDGES = 320000
D = 128

NUM_CORES = 2
NUM_SUBCORES = 16
NUM_WORKERS = NUM_CORES * NUM_SUBCORES
BLK = 80
NBLK = 126
EDGES_PAD = NUM_WORKERS * NBLK * BLK
ACC_ROWS = 10112
ROWS_PER_TILE = ACC_ROWS // NUM_SUBCORES


def _sc_agg_body(h_hbm, src_hbm, dst_hbm, zeros_hbm, out_hbm,
                 si0, si1, si2, si3, di0, di1, di2, di3, rows0, rows1,
                 acc, g0, g1, sc0, sc1, is0, is1, is2, is3):
    cid = lax.axis_index("c")
    sid = lax.axis_index("s")
    wid = cid * NUM_SUBCORES + sid
    base0 = wid * (NBLK * BLK)

    sidx = (si0, si1, si2, si3)
    didx = (di0, di1, di2, di3)
    isem = (is0, is1, is2, is3)
    rows = (rows0, rows1)
    g = (g0, g1)
    sc = (sc0, sc1)

    def idx_load(j, k):
        pltpu.async_copy(src_hbm.at[pl.ds(base0 + j * BLK, BLK)], sidx[k], isem[k])
        pltpu.async_copy(dst_hbm.at[pl.ds(base0 + j * BLK, BLK)], didx[k], isem[k])

    def idx_wait(j, k):
        pltpu.make_async_copy(
            src_hbm.at[pl.ds(base0 + j * BLK, BLK)], sidx[k], isem[k]).wait()
        pltpu.make_async_copy(
            dst_hbm.at[pl.ds(base0 + j * BLK, BLK)], didx[k], isem[k]).wait()

    def gather(k, p):
        pltpu.async_copy(h_hbm.at[sidx[k]], rows[p], g[p])

    def gather_wait(k, p):
        pltpu.make_async_copy(h_hbm.at[sidx[k]], rows[p], g[p]).wait()

    def scatter(k, p):
        pltpu.async_copy(rows[p], acc.at[didx[k]], sc[p], add=True)

    def scatter_wait(k, p):
        pltpu.make_async_copy(rows[p], acc.at[didx[k]], sc[p]).wait()

    def body(j, k, first=False, load_idx=True, next_gather=True, last=False):
        p = k & 1
        gather_wait(k, p)
        scatter(k, p)
        if not first:
            scatter_wait((k - 1) & 3, 1 - p)
        if load_idx:
            idx_load(j + 3, (k - 1) & 3)
        if next_gather:
            idx_wait(j + 1, (k + 1) & 3)
            gather((k + 1) & 3, 1 - p)
        if last:
            scatter_wait(k, p)

    idx_load(0, 0)
    idx_load(1, 1)
    idx_load(2, 2)
    pltpu.sync_copy(zeros_hbm, acc.at[pl.ds(sid * ROWS_PER_TILE, ROWS_PER_TILE)])
    idx_wait(0, 0)
    plsc.subcore_barrier()
    gather(0, 0)

    body(0, 0, first=True)
    body(1, 1)
    body(2, 2)
    body(3, 3)

    @pl.loop(4, NBLK - 6, step=4)
    def _(j):
        body(j, 0)
        body(j + 1, 1)
        body(j + 2, 2)
        body(j + 3, 3)

    body(NBLK - 6, 0)
    body(NBLK - 5, 1)
    body(NBLK - 4, 2)
    body(NBLK - 3, 3, load_idx=False)
    body(NBLK - 2, 0, load_idx=False)
    body(NBLK - 1, 1, load_idx=False, next_gather=False, last=True)

    plsc.subcore_barrier()
    row0 = sid * ROWS_PER_TILE
    pltpu.sync_copy(acc.at[pl.ds(row0, ROWS_PER_TILE)],
                    out_hbm.at[cid, pl.ds(row0, ROWS_PER_TILE)])


@jax.jit
def _sc_agg(h, src, dst, zeros):
    mesh = plsc.VectorSubcoreMesh(core_axis_name="c", subcore_axis_name="s")
    k = pl.kernel(
        _sc_agg_body,
        out_type=jax.ShapeDtypeStruct((NUM_CORES, ACC_ROWS, D), jnp.float32),
        mesh=mesh,
        scratch_types=[
            pltpu.VMEM((BLK,), jnp.int32),
            pltpu.VMEM((BLK,), jnp.int32),
            pltpu.VMEM((BLK,), jnp.int32),
            pltpu.VMEM((BLK,), jnp.int32),
            pltpu.VMEM((BLK,), jnp.int32),
            pltpu.VMEM((BLK,), jnp.int32),
            pltpu.VMEM((BLK,), jnp.int32),
            pltpu.VMEM((BLK,), jnp.int32),
            pltpu.VMEM((BLK, D), jnp.float32),
            pltpu.VMEM((BLK, D), jnp.float32),
            pltpu.VMEM_SHARED((ACC_ROWS, D), jnp.float32),
            pltpu.SemaphoreType.DMA,
            pltpu.SemaphoreType.DMA,
            pltpu.SemaphoreType.DMA,
            pltpu.SemaphoreType.DMA,
            pltpu.SemaphoreType.DMA,
            pltpu.SemaphoreType.DMA,
            pltpu.SemaphoreType.DMA,
            pltpu.SemaphoreType.DMA,
        ],
    )
    return k(h, src, dst, zeros)


ROW_BLK = 1000


def _tc_mlp_body(x_ref, p_ref, wt_ref, b_ref, o_ref, *, relu):
    s = x_ref[...] + p_ref[0] + p_ref[1]
    y = jnp.dot(s, wt_ref[...], preferred_element_type=jnp.float32) + b_ref[...]
    if relu:
        y = jnp.maximum(y, 0.0)
    o_ref[...] = y


def _tc_mlp(x, parts, wt, b, relu):
    grid = (N_NODES // ROW_BLK,)
    return pl.pallas_call(
        functools.partial(_tc_mlp_body, relu=relu),
        grid=grid,
        in_specs=[
            pl.BlockSpec((ROW_BLK, D), lambda i: (i, 0)),
            pl.BlockSpec((NUM_CORES, ROW_BLK, D), lambda i: (0, i, 0)),
            pl.BlockSpec((D, D), lambda i: (0, 0)),
            pl.BlockSpec((1, D), lambda i: (0, 0)),
        ],
        out_specs=pl.BlockSpec((ROW_BLK, D), lambda i: (i, 0)),
        out_shape=jax.ShapeDtypeStruct((N_NODES, D), jnp.float32),
    )(x, parts, wt, b)


def kernel(x, edge_index, W1, b1, W2, b2):
    npad = EDGES_PAD - N_EDGES
    src = jnp.concatenate(
        [edge_index[0].astype(jnp.int32), jnp.zeros((npad,), jnp.int32)])
    dst = jnp.concatenate(
        [edge_index[1].astype(jnp.int32),
         jnp.full((npad,), ACC_ROWS - 1, jnp.int32)])
    zeros = jnp.zeros((ROWS_PER_TILE, D), jnp.float32)

    agg1 = _sc_agg(x, src, dst, zeros)
    h = _tc_mlp(x, agg1, W1.T, b1.reshape(1, D), relu=True)
    agg2 = _sc_agg(h, src, dst, zeros)
    out = _tc_mlp(h, agg2, W2.T, b2.reshape(1, D), relu=False)
    return out
